# baseline (speedup 1.0000x reference)
import functools

import jax
import jax.numpy as jnp
from jax import lax
from jax.experimental import pallas as pl
from jax.experimental.pallas import tpu as pltpu
from jax.experimental.pallas import tpu_sc as plsc

NUM_FIELDS = 26
EMBED_DIM = 16
FIELD_SIZE = 100000

NC = 2
NS = 16
LANES = 16


@functools.partial(jax.jit, static_argnames=("batch",))
def _embedding_gather(x_flat, table, *, batch):
    B = batch * NUM_FIELDS
    NW = NC * NS
    b_per_w = B // NW
    C = 1664
    nch = b_per_w // C

    mesh = plsc.VectorSubcoreMesh(core_axis_name="c", subcore_axis_name="s")

    @functools.partial(
        pl.kernel,
        out_type=jax.ShapeDtypeStruct((B, EMBED_DIM), jnp.float32),
        mesh=mesh,
        scratch_types=[
            pltpu.VMEM((C,), jnp.int32),
            pltpu.VMEM((C, EMBED_DIM), jnp.float32),
            pltpu.SemaphoreType.DMA,
        ],
        compiler_params=pltpu.CompilerParams(use_tc_tiling_on_sc=False),
    )
    def k(x_hbm, table_hbm, out_hbm, idx_v, rows_v, sem):
        wid = lax.axis_index("s") * NC + lax.axis_index("c")
        wbase = wid * b_per_w

        def chunk_body(c, _):
            base = wbase + c * C
            pltpu.sync_copy(x_hbm.at[pl.ds(base, C)], idx_v)

            def add_body(j, _):
                s = j * LANES
                pos = s + lax.iota(jnp.int32, LANES)
                f = lax.rem(pos, NUM_FIELDS)
                idx_v[pl.ds(s, LANES)] = idx_v[pl.ds(s, LANES)] + f * FIELD_SIZE
                return 0

            lax.fori_loop(0, C // LANES, add_body, 0)
            pltpu.async_copy(table_hbm.at[idx_v], rows_v, sem).wait()
            pltpu.sync_copy(rows_v, out_hbm.at[pl.ds(base, C)])
            return 0

        lax.fori_loop(0, nch, chunk_body, 0)

    return k(x_flat, table)


def kernel(x, table):
    batch = x.shape[0]
    x_flat = x.reshape(batch * NUM_FIELDS)
    out = _embedding_gather(x_flat, table, batch=batch)
    return out.reshape(batch, NUM_FIELDS, EMBED_DIM)


# speedup vs baseline: 1.0066x; 1.0066x over previous
import functools

import jax
import jax.numpy as jnp
from jax import lax
from jax.experimental import pallas as pl
from jax.experimental.pallas import tpu as pltpu
from jax.experimental.pallas import tpu_sc as plsc

NUM_FIELDS = 26
EMBED_DIM = 16
FIELD_SIZE = 100000

NC = 2
NS = 16
LANES = 16


@functools.partial(jax.jit, static_argnames=("batch",))
def _embedding_gather(x_flat, table, *, batch):
    B = batch * NUM_FIELDS
    NW = NC * NS
    b_per_w = B // NW
    C = 1664
    nch = b_per_w // C
    NBUF = 4

    mesh = plsc.VectorSubcoreMesh(core_axis_name="c", subcore_axis_name="s")

    @functools.partial(
        pl.kernel,
        out_type=jax.ShapeDtypeStruct((B, EMBED_DIM), jnp.float32),
        mesh=mesh,
        scratch_types=[
            pltpu.VMEM((b_per_w,), jnp.int32),
            pltpu.VMEM((NBUF, C, EMBED_DIM), jnp.float32),
            [pltpu.SemaphoreType.DMA] * NBUF,
            [pltpu.SemaphoreType.DMA] * NBUF,
        ],
        compiler_params=pltpu.CompilerParams(use_tc_tiling_on_sc=False),
    )
    def k(x_hbm, table_hbm, out_hbm, idx_all, rows, gsems, osems):
        wid = lax.axis_index("s") * NC + lax.axis_index("c")
        wbase = wid * b_per_w

        pltpu.sync_copy(x_hbm.at[pl.ds(wbase, b_per_w)], idx_all)

        def add_body(j, _):
            s = j * LANES
            pos = s + lax.iota(jnp.int32, LANES)
            f = lax.rem(pos, NUM_FIELDS)
            idx_all[pl.ds(s, LANES)] = idx_all[pl.ds(s, LANES)] + f * FIELD_SIZE
            return 0

        lax.fori_loop(0, b_per_w // LANES, add_body, 0)

        def gather(c, b):
            return pltpu.async_copy(
                table_hbm.at[idx_all.at[pl.ds(c * C, C)]], rows.at[b], gsems[b]
            )

        gcopies = {}
        ocopies = {}
        for c in range(min(NBUF, nch)):
            gcopies[c] = gather(c, c)
        for c in range(nch):
            b = c % NBUF
            gcopies.pop(c).wait()
            ocopies[c] = pltpu.async_copy(
                rows.at[b], out_hbm.at[pl.ds(wbase + c * C, C)], osems[b]
            )
            nxt = c + NBUF
            if nxt < nch:
                ocopies.pop(c).wait()
                gcopies[nxt] = gather(nxt, b)
        for c in list(ocopies):
            ocopies.pop(c).wait()

    return k(x_flat, table)


def kernel(x, table):
    batch = x.shape[0]
    x_flat = x.reshape(batch * NUM_FIELDS)
    out = _embedding_gather(x_flat, table, batch=batch)
    return out.reshape(batch, NUM_FIELDS, EMBED_DIM)


# speedup vs baseline: 1.2870x; 1.2785x over previous
import functools

import jax
import jax.numpy as jnp
from jax import lax
from jax.experimental import pallas as pl
from jax.experimental.pallas import tpu as pltpu
from jax.experimental.pallas import tpu_sc as plsc

NUM_FIELDS = 26
EMBED_DIM = 16
FIELD_SIZE = 100000

NC = 2
NS = 16
LANES = 16


@functools.partial(jax.jit, static_argnames=("batch",))
def _embedding_gather(xT, table, *, batch):
    NW = NC * NS
    BB = batch // NW

    mesh = plsc.VectorSubcoreMesh(core_axis_name="c", subcore_axis_name="s")

    @functools.partial(
        pl.kernel,
        out_type=jax.ShapeDtypeStruct((NUM_FIELDS, EMBED_DIM, batch), jnp.float32),
        mesh=mesh,
        scratch_types=[
            [pltpu.VMEM((BB,), jnp.int32)] * 4,
            [pltpu.VMEM((BB, EMBED_DIM), jnp.float32)] * 2,
            [pltpu.VMEM((EMBED_DIM, BB), jnp.float32)] * 2,
            [pltpu.SemaphoreType.DMA] * 4,
            [pltpu.SemaphoreType.DMA] * 2,
            [pltpu.SemaphoreType.DMA] * 2,
        ],
        compiler_params=pltpu.CompilerParams(
            use_tc_tiling_on_sc=False, needs_layout_passes=False
        ),
    )
    def k(xT_hbm, table_hbm, out_hbm, idxs, rows, outs, xsems, gsems, osems):
        wid = lax.axis_index("s") * NC + lax.axis_index("c")
        b0 = wid * BB

        def stage_x(f):
            return pltpu.async_copy(
                xT_hbm.at[f, pl.ds(b0, BB)], idxs[f % 4], xsems[f % 4]
            )

        def add_offsets(f):
            buf = idxs[f % 4]

            def body(j, _):
                s = j * LANES
                buf[pl.ds(s, LANES)] = buf[pl.ds(s, LANES)] + f * FIELD_SIZE
                return 0

            lax.fori_loop(0, BB // LANES, body, 0)

        def fire_gather(f):
            return pltpu.async_copy(
                table_hbm.at[idxs[f % 4]], rows[f % 2], gsems[f % 2]
            )

        def transpose(f):
            src = rows[f % 2]
            dst = outs[f % 2]

            def body(j, _):
                rvec = j * LANES + lax.iota(jnp.int32, LANES)
                for e in range(EMBED_DIM):
                    cvec = jnp.full((LANES,), e, dtype=jnp.int32)
                    dst[e, pl.ds(j * LANES, LANES)] = plsc.load_gather(
                        src, [rvec, cvec]
                    )
                return 0

            lax.fori_loop(0, BB // LANES, body, 0)

        def fire_out(f):
            return pltpu.async_copy(
                outs[f % 2], out_hbm.at[f, :, pl.ds(b0, BB)], osems[f % 2]
            )

        xcopies, gcopies, ocopies = {}, {}, {}
        for f in range(3):
            xcopies[f] = stage_x(f)
        for f in range(NUM_FIELDS):
            xcopies.pop(f).wait()
            add_offsets(f)
            gcopies[f] = fire_gather(f)
            if f >= 1:
                gcopies.pop(f - 1).wait()
                transpose(f - 1)
                if f - 3 >= 0:
                    ocopies.pop(f - 3).wait()
                ocopies[f - 1] = fire_out(f - 1)
            if f + 3 < NUM_FIELDS:
                xcopies[f + 3] = stage_x(f + 3)
        gcopies.pop(NUM_FIELDS - 1).wait()
        transpose(NUM_FIELDS - 1)
        ocopies[NUM_FIELDS - 1] = fire_out(NUM_FIELDS - 1)
        for f in list(ocopies):
            ocopies.pop(f).wait()

    return k(xT, table)


def kernel(x, table):
    batch = x.shape[0]
    out_view = _embedding_gather(x.T, table, batch=batch)
    return jnp.transpose(out_view, (2, 0, 1))
